# baseline (speedup 1.0000x reference)
import functools

import jax
import jax.numpy as jnp
from jax import lax
from jax.experimental import pallas as pl
from jax.experimental.pallas import tpu as pltpu
from jax.experimental.pallas import tpu_sc as plsc

N = 10000
E = 320000
F = 128
F2 = F // 2
NC = 2
NS = 16
NW = NC * NS
EW = E // NW
CH = 125
KT = (E // NS) // CH
RPT = N // NS
RB = 2000
NRB = N // RB


@functools.cache
def _deg_kernel_fn():
    return functools.partial(
        pl.kernel,
        out_type=jax.ShapeDtypeStruct((NW * N,), jnp.float32),
        scratch_types=[
            pltpu.VMEM((EW,), jnp.int32),
            pltpu.VMEM((N,), jnp.float32),
        ],
        mesh=plsc.VectorSubcoreMesh(core_axis_name="c", subcore_axis_name="s",
                                    num_cores=NC, num_subcores=NS),
        compiler_params=pltpu.CompilerParams(needs_layout_passes=False,
                                             use_tc_tiling_on_sc=False),
    )(_deg_body)


def _deg_body(src_hbm, out_hbm, src_v, hist_v):
    c = lax.axis_index("c")
    s = lax.axis_index("s")
    wid = c * NS + s
    pltpu.sync_copy(src_hbm.at[pl.ds(wid * EW, EW)], src_v)

    zeros16 = jnp.zeros((16,), jnp.float32)

    @pl.loop(0, N // 16)
    def _zero(i):
        hist_v[pl.ds(i * 16, 16)] = zeros16

    ones16 = jnp.ones((16,), jnp.float32)

    @pl.loop(0, EW // 16)
    def _accum(i):
        idx = src_v[pl.ds(i * 16, 16)]
        plsc.addupdate_scatter(hist_v, [idx], ones16)

    pltpu.sync_copy(hist_v, out_hbm.at[pl.ds(wid * N, N)])


def _mm_body(feat_ref, w_ref, degp_ref, hs0_ref, hs1_ref, norm_ref):
    deg = jnp.sum(degp_ref[:, 0, 0, :], axis=0)
    norm = lax.rsqrt(jnp.maximum(deg, 1.0))
    h = jnp.dot(feat_ref[...], w_ref[...], preferred_element_type=jnp.float32)
    hs = h * norm[:, None]
    hs0_ref[...] = hs[:, :F2]
    hs1_ref[...] = hs[:, F2:]
    norm_ref[...] = norm.reshape(1, 1, RB)


def _matmul_norm(feat, weight, degp):
    return pl.pallas_call(
        _mm_body,
        grid=(NRB,),
        in_specs=[
            pl.BlockSpec((RB, F), lambda i: (i, 0)),
            pl.BlockSpec((F, F), lambda i: (0, 0)),
            pl.BlockSpec((NW, 1, 1, RB), lambda i: (0, i, 0, 0)),
        ],
        out_specs=[
            pl.BlockSpec((RB, F2), lambda i: (i, 0)),
            pl.BlockSpec((RB, F2), lambda i: (i, 0)),
            pl.BlockSpec((1, 1, RB), lambda i: (i, 0, 0)),
        ],
        out_shape=[
            jax.ShapeDtypeStruct((N, F2), jnp.float32),
            jax.ShapeDtypeStruct((N, F2), jnp.float32),
            jax.ShapeDtypeStruct((NRB, 1, RB), jnp.float32),
        ],
    )(feat, weight, degp)


@functools.cache
def _scatter_kernel_fn():
    return functools.partial(
        pl.kernel,
        out_type=jax.ShapeDtypeStruct((N, F), jnp.float32),
        scratch_types=[
            pltpu.VMEM((KT, CH), jnp.int32),
            pltpu.VMEM((KT, CH), jnp.int32),
            pltpu.VMEM((4, CH, F2), jnp.float32),
            pltpu.VMEM((N,), jnp.float32),
            pltpu.VMEM((F2,), jnp.float32),
            pltpu.VMEM_SHARED((N, F2), jnp.float32),
            pltpu.SemaphoreType.DMA,
            pltpu.SemaphoreType.DMA,
            pltpu.SemaphoreType.DMA,
            pltpu.SemaphoreType.DMA,
            pltpu.SemaphoreType.DMA,
            pltpu.SemaphoreType.DMA,
            pltpu.SemaphoreType.DMA,
            pltpu.SemaphoreType.DMA,
        ],
        mesh=plsc.VectorSubcoreMesh(core_axis_name="c", subcore_axis_name="s",
                                    num_cores=NC, num_subcores=NS),
        compiler_params=pltpu.CompilerParams(needs_layout_passes=False,
                                             use_tc_tiling_on_sc=False),
    )(_scatter_body)


def _scatter_body(hs0_hbm, hs1_hbm, src_hbm, dst_hbm, zeros_hbm,
                  norm_hbm, bias_hbm, out_hbm,
                  src_v, dst_v, rows_v, norm_v, bias_v, acc_s,
                  gs0, gs1, gs2, gs3, ss0, ss1, ss2, ss3):
    c = lax.axis_index("c")
    s = lax.axis_index("s")
    gsem = (gs0, gs1, gs2, gs3)
    ssem = (ss0, ss1, ss2, ss3)

    pltpu.sync_copy(zeros_hbm, acc_s.at[pl.ds(s * RPT, RPT)])
    pltpu.sync_copy(src_hbm.at[pl.ds(s * KT, KT)], src_v)
    pltpu.sync_copy(dst_hbm.at[pl.ds(s * KT, KT)], dst_v)
    pltpu.sync_copy(norm_hbm, norm_v)
    pltpu.sync_copy(bias_hbm.at[pl.ds(c * F2, F2)], bias_v)
    plsc.subcore_barrier()

    def _edge_loop(hs_hbm):
        def _gather(j, b):
            pltpu.async_copy(hs_hbm.at[src_v.at[j]], rows_v.at[b], gsem[b])

        def _scatter(j, b):
            pltpu.async_copy(rows_v.at[b], acc_s.at[dst_v.at[j]], ssem[b],
                             add=True)

        _gather(0, 0)

        @pl.loop(0, KT, step=4)
        def _edges(j0):
            for b in range(4):
                j = j0 + b
                pltpu.make_async_copy(hs_hbm.at[src_v.at[j]], rows_v.at[b],
                                      gsem[b]).wait()
                _scatter(j, b)
                b1 = (b + 1) % 4

                @pl.when(j >= 3)
                def _drain_scatter():
                    pltpu.make_async_copy(rows_v.at[b1],
                                          acc_s.at[dst_v.at[j]],
                                          ssem[b1]).wait()

                @pl.when(j + 1 < KT)
                def _start_next():
                    _gather(j + 1, b1)

        for t in (KT - 3, KT - 2, KT - 1):
            pltpu.make_async_copy(rows_v.at[t % 4], acc_s.at[dst_v.at[0]],
                                  ssem[t % 4]).wait()

    @pl.when(c == 0)
    def _c0():
        _edge_loop(hs0_hbm)

    @pl.when(c == 1)
    def _c1():
        _edge_loop(hs1_hbm)

    plsc.subcore_barrier()

    r0 = s * RPT
    biases = [bias_v[pl.ds(k * 16, 16)] for k in range(F2 // 16)]
    for p in range(RPT // CH):
        buf = rows_v.at[p % 4]
        base = r0 + p * CH
        pltpu.sync_copy(acc_s.at[pl.ds(base, CH)], buf)

        @pl.loop(0, CH)
        def _rows(r):
            ridx = jnp.zeros((16,), jnp.int32) + (base + r)
            nrm = plsc.load_gather(norm_v, [ridx])
            for k in range(F2 // 16):
                v = buf[r, pl.ds(k * 16, 16)]
                buf[r, pl.ds(k * 16, 16)] = v * nrm + biases[k]

        pltpu.sync_copy(buf, out_hbm.at[pl.ds(base, CH), pl.ds(c * F2, F2)])


def kernel(feat, edge_index, weight, bias):
    src = edge_index[0]
    dst = edge_index[1]
    src2d = src.reshape(NS * KT, CH)
    dst2d = dst.reshape(NS * KT, CH)
    zeros = jnp.zeros((RPT, F2), jnp.float32)

    degp = _deg_kernel_fn()(src).reshape(NW, NRB, 1, RB)
    hs0, hs1, norm3 = _matmul_norm(feat, weight, degp)
    return _scatter_kernel_fn()(hs0, hs1, src2d, dst2d, zeros,
                                norm3.reshape(N), bias)


# speedup vs baseline: 1.2827x; 1.2827x over previous
import functools

import jax
import jax.numpy as jnp
from jax import lax
from jax.experimental import pallas as pl
from jax.experimental.pallas import tpu as pltpu
from jax.experimental.pallas import tpu_sc as plsc

N = 10000
E = 320000
F = 128
F2 = F // 2
NC = 2
NS = 16
NW = NC * NS
EW = E // NW
CH = 125
KT = (E // NS) // CH
RPT = N // NS
RB = 2000
NRB = N // RB


@functools.cache
def _deg_kernel_fn():
    return functools.partial(
        pl.kernel,
        out_type=jax.ShapeDtypeStruct((NW * N,), jnp.float32),
        scratch_types=[
            pltpu.VMEM((EW,), jnp.int32),
            pltpu.VMEM((N,), jnp.float32),
        ],
        mesh=plsc.VectorSubcoreMesh(core_axis_name="c", subcore_axis_name="s",
                                    num_cores=NC, num_subcores=NS),
        compiler_params=pltpu.CompilerParams(needs_layout_passes=False,
                                             use_tc_tiling_on_sc=False),
    )(_deg_body)


def _deg_body(src_hbm, out_hbm, src_v, hist_v):
    c = lax.axis_index("c")
    s = lax.axis_index("s")
    wid = c * NS + s
    pltpu.sync_copy(src_hbm.at[pl.ds(wid * EW, EW)], src_v)

    zeros16 = jnp.zeros((16,), jnp.float32)

    @pl.loop(0, N // 16)
    def _zero(i):
        hist_v[pl.ds(i * 16, 16)] = zeros16

    ones16 = jnp.ones((16,), jnp.float32)

    @pl.loop(0, EW // 16)
    def _accum(i):
        idx = src_v[pl.ds(i * 16, 16)]
        plsc.addupdate_scatter(hist_v, [idx], ones16)

    pltpu.sync_copy(hist_v, out_hbm.at[pl.ds(wid * N, N)])


def _mm_body(feat_ref, w_ref, degp_ref, hs0_ref, hs1_ref, norm_ref):
    deg = jnp.sum(degp_ref[:, 0, 0, :], axis=0)
    norm = lax.rsqrt(jnp.maximum(deg, 1.0))
    h = jnp.dot(feat_ref[...], w_ref[...], preferred_element_type=jnp.float32)
    hs = h * norm[:, None]
    hs0_ref[...] = hs[:, :F2]
    hs1_ref[...] = hs[:, F2:]
    norm_ref[...] = norm.reshape(1, 1, RB)


def _matmul_norm(feat, weight, degp):
    return pl.pallas_call(
        _mm_body,
        grid=(NRB,),
        in_specs=[
            pl.BlockSpec((RB, F), lambda i: (i, 0)),
            pl.BlockSpec((F, F), lambda i: (0, 0)),
            pl.BlockSpec((NW, 1, 1, RB), lambda i: (0, i, 0, 0)),
        ],
        out_specs=[
            pl.BlockSpec((RB, F2), lambda i: (i, 0)),
            pl.BlockSpec((RB, F2), lambda i: (i, 0)),
            pl.BlockSpec((1, 1, RB), lambda i: (i, 0, 0)),
        ],
        out_shape=[
            jax.ShapeDtypeStruct((N, F2), jnp.float32),
            jax.ShapeDtypeStruct((N, F2), jnp.float32),
            jax.ShapeDtypeStruct((NRB, 1, RB), jnp.float32),
        ],
    )(feat, weight, degp)


@functools.cache
def _scatter_kernel_fn():
    return functools.partial(
        pl.kernel,
        out_type=jax.ShapeDtypeStruct((N, F), jnp.float32),
        scratch_types=[
            pltpu.VMEM((KT, CH), jnp.int32),
            pltpu.VMEM((KT, CH), jnp.int32),
            pltpu.VMEM((4, CH, F2), jnp.float32),
            pltpu.VMEM((N,), jnp.float32),
            pltpu.VMEM((F2,), jnp.float32),
            pltpu.VMEM_SHARED((N, F2), jnp.float32),
            pltpu.SemaphoreType.DMA,
            pltpu.SemaphoreType.DMA,
            pltpu.SemaphoreType.DMA,
            pltpu.SemaphoreType.DMA,
            pltpu.SemaphoreType.DMA,
            pltpu.SemaphoreType.DMA,
            pltpu.SemaphoreType.DMA,
            pltpu.SemaphoreType.DMA,
        ],
        mesh=plsc.VectorSubcoreMesh(core_axis_name="c", subcore_axis_name="s",
                                    num_cores=NC, num_subcores=NS),
        compiler_params=pltpu.CompilerParams(needs_layout_passes=False,
                                             use_tc_tiling_on_sc=False),
    )(_scatter_body)


def _scatter_body(hs0_hbm, hs1_hbm, src_hbm, dst_hbm, zeros_hbm,
                  norm_hbm, bias_hbm, out_hbm,
                  src_v, dst_v, rows_v, norm_v, bias_v, acc_s,
                  gs0, gs1, gs2, gs3, ss0, ss1, ss2, ss3):
    c = lax.axis_index("c")
    s = lax.axis_index("s")
    gsem = (gs0, gs1, gs2, gs3)
    ssem = (ss0, ss1, ss2, ss3)

    pltpu.sync_copy(zeros_hbm, acc_s.at[pl.ds(s * RPT, RPT)])
    pltpu.sync_copy(src_hbm.at[pl.ds(s * KT, KT)], src_v)
    pltpu.sync_copy(dst_hbm.at[pl.ds(s * KT, KT)], dst_v)
    pltpu.sync_copy(norm_hbm, norm_v)
    pltpu.sync_copy(bias_hbm.at[pl.ds(c * F2, F2)], bias_v)
    plsc.subcore_barrier()

    def _edge_loop(hs_hbm):
        def _gather(j, b):
            pltpu.async_copy(hs_hbm.at[src_v.at[j]], rows_v.at[b], gsem[b])

        def _scatter(j, b):
            pltpu.async_copy(rows_v.at[b], acc_s.at[dst_v.at[j]], ssem[b],
                             add=True)

        _gather(0, 0)
        _gather(1, 1)

        @pl.loop(0, KT, step=4)
        def _edges(j0):
            for b in range(4):
                j = j0 + b
                pltpu.make_async_copy(hs_hbm.at[src_v.at[j]], rows_v.at[b],
                                      gsem[b]).wait()
                _scatter(j, b)
                b2 = (b + 2) % 4

                @pl.when(j >= 2)
                def _drain_scatter():
                    pltpu.make_async_copy(rows_v.at[b2],
                                          acc_s.at[dst_v.at[j]],
                                          ssem[b2]).wait()

                @pl.when(j + 2 < KT)
                def _start_next():
                    _gather(j + 2, b2)

        for t in (KT - 2, KT - 1):
            pltpu.make_async_copy(rows_v.at[t % 4], acc_s.at[dst_v.at[0]],
                                  ssem[t % 4]).wait()

    @pl.when(c == 0)
    def _c0():
        _edge_loop(hs0_hbm)

    @pl.when(c == 1)
    def _c1():
        _edge_loop(hs1_hbm)

    plsc.subcore_barrier()

    r0 = s * RPT
    biases = [bias_v[pl.ds(k * 16, 16)] for k in range(F2 // 16)]
    for p in range(RPT // CH):
        buf = rows_v.at[p % 4]
        base = r0 + p * CH
        pltpu.sync_copy(acc_s.at[pl.ds(base, CH)], buf)

        @pl.loop(0, CH)
        def _rows(r):
            ridx = jnp.zeros((16,), jnp.int32) + (base + r)
            nrm = plsc.load_gather(norm_v, [ridx])
            for k in range(F2 // 16):
                v = buf[r, pl.ds(k * 16, 16)]
                buf[r, pl.ds(k * 16, 16)] = v * nrm + biases[k]

        pltpu.sync_copy(buf, out_hbm.at[pl.ds(base, CH), pl.ds(c * F2, F2)])


def kernel(feat, edge_index, weight, bias):
    src = edge_index[0]
    dst = edge_index[1]
    src2d = src.reshape(NS * KT, CH)
    dst2d = dst.reshape(NS * KT, CH)
    zeros = jnp.zeros((RPT, F2), jnp.float32)

    degp = _deg_kernel_fn()(src).reshape(NW, NRB, 1, RB)
    hs0, hs1, norm3 = _matmul_norm(feat, weight, degp)
    return _scatter_kernel_fn()(hs0, hs1, src2d, dst2d, zeros,
                                norm3.reshape(N), bias)


# speedup vs baseline: 1.3182x; 1.0277x over previous
import functools

import jax
import jax.numpy as jnp
from jax import lax
from jax.experimental import pallas as pl
from jax.experimental.pallas import tpu as pltpu
from jax.experimental.pallas import tpu_sc as plsc

N = 10000
E = 320000
F = 128
F2 = F // 2
NC = 2
NS = 16
NW = NC * NS
EW = E // NW
CH = 125
KT = (E // NS) // CH
RPT = N // NS
RB = 2000
NRB = N // RB


@functools.cache
def _deg_kernel_fn():
    return functools.partial(
        pl.kernel,
        out_type=jax.ShapeDtypeStruct((NW * N,), jnp.float32),
        scratch_types=[
            pltpu.VMEM((EW,), jnp.int32),
            pltpu.VMEM((N,), jnp.float32),
            pltpu.SemaphoreType.DMA,
        ],
        mesh=plsc.VectorSubcoreMesh(core_axis_name="c", subcore_axis_name="s",
                                    num_cores=NC, num_subcores=NS),
        compiler_params=pltpu.CompilerParams(needs_layout_passes=False,
                                             use_tc_tiling_on_sc=False),
    )(_deg_body)


def _deg_body(src_hbm, out_hbm, src_v, hist_v, sem):
    c = lax.axis_index("c")
    s = lax.axis_index("s")
    wid = c * NS + s
    stage = pltpu.async_copy(src_hbm.at[pl.ds(wid * EW, EW)], src_v, sem)

    zeros16 = jnp.zeros((16,), jnp.float32)

    @pl.loop(0, N // 16, unroll=8)
    def _zero(i):
        hist_v[pl.ds(i * 16, 16)] = zeros16

    stage.wait()
    ones16 = jnp.ones((16,), jnp.float32)

    @pl.loop(0, EW // 16, unroll=8)
    def _accum(i):
        idx = src_v[pl.ds(i * 16, 16)]
        plsc.addupdate_scatter(hist_v, [idx], ones16)

    pltpu.sync_copy(hist_v, out_hbm.at[pl.ds(wid * N, N)])


def _mm_body(feat_ref, w_ref, degp_ref, hs0_ref, hs1_ref, norm_ref):
    deg = jnp.sum(degp_ref[:, 0, 0, :], axis=0)
    norm = lax.rsqrt(jnp.maximum(deg, 1.0))
    h = jnp.dot(feat_ref[...], w_ref[...], preferred_element_type=jnp.float32)
    hs = h * norm[:, None]
    hs0_ref[...] = hs[:, :F2]
    hs1_ref[...] = hs[:, F2:]
    norm_ref[...] = norm.reshape(1, 1, RB)


def _matmul_norm(feat, weight, degp):
    return pl.pallas_call(
        _mm_body,
        grid=(NRB,),
        in_specs=[
            pl.BlockSpec((RB, F), lambda i: (i, 0)),
            pl.BlockSpec((F, F), lambda i: (0, 0)),
            pl.BlockSpec((NW, 1, 1, RB), lambda i: (0, i, 0, 0)),
        ],
        out_specs=[
            pl.BlockSpec((RB, F2), lambda i: (i, 0)),
            pl.BlockSpec((RB, F2), lambda i: (i, 0)),
            pl.BlockSpec((1, 1, RB), lambda i: (i, 0, 0)),
        ],
        out_shape=[
            jax.ShapeDtypeStruct((N, F2), jnp.float32),
            jax.ShapeDtypeStruct((N, F2), jnp.float32),
            jax.ShapeDtypeStruct((NRB, 1, RB), jnp.float32),
        ],
    )(feat, weight, degp)


@functools.cache
def _scatter_kernel_fn():
    return functools.partial(
        pl.kernel,
        out_type=jax.ShapeDtypeStruct((N, F), jnp.float32),
        scratch_types=[
            pltpu.VMEM((KT, CH), jnp.int32),
            pltpu.VMEM((KT, CH), jnp.int32),
            pltpu.VMEM((4, CH, F2), jnp.float32),
            pltpu.VMEM((N,), jnp.float32),
            pltpu.VMEM((F2,), jnp.float32),
            pltpu.VMEM_SHARED((N, F2), jnp.float32),
            pltpu.SemaphoreType.DMA,
            pltpu.SemaphoreType.DMA,
            pltpu.SemaphoreType.DMA,
            pltpu.SemaphoreType.DMA,
            pltpu.SemaphoreType.DMA,
            pltpu.SemaphoreType.DMA,
            pltpu.SemaphoreType.DMA,
            pltpu.SemaphoreType.DMA,
        ],
        mesh=plsc.VectorSubcoreMesh(core_axis_name="c", subcore_axis_name="s",
                                    num_cores=NC, num_subcores=NS),
        compiler_params=pltpu.CompilerParams(needs_layout_passes=False,
                                             use_tc_tiling_on_sc=False),
    )(_scatter_body)


def _scatter_body(hs0_hbm, hs1_hbm, src_hbm, dst_hbm, zeros_hbm,
                  norm_hbm, bias_hbm, out_hbm,
                  src_v, dst_v, rows_v, norm_v, bias_v, acc_s,
                  gs0, gs1, gs2, gs3, ss0, ss1, ss2, ss3):
    c = lax.axis_index("c")
    s = lax.axis_index("s")
    gsem = (gs0, gs1, gs2, gs3)
    ssem = (ss0, ss1, ss2, ss3)

    stages = [
        pltpu.async_copy(zeros_hbm, acc_s.at[pl.ds(s * RPT, RPT)], ssem[0]),
        pltpu.async_copy(src_hbm.at[pl.ds(s * KT, KT)], src_v, gsem[0]),
        pltpu.async_copy(dst_hbm.at[pl.ds(s * KT, KT)], dst_v, gsem[1]),
        pltpu.async_copy(norm_hbm, norm_v, gsem[2]),
        pltpu.async_copy(bias_hbm.at[pl.ds(c * F2, F2)], bias_v, gsem[3]),
    ]
    for d in stages:
        d.wait()
    plsc.subcore_barrier()

    def _edge_loop(hs_hbm):
        def _gather(j, b):
            pltpu.async_copy(hs_hbm.at[src_v.at[j]], rows_v.at[b], gsem[b])

        def _scatter(j, b):
            pltpu.async_copy(rows_v.at[b], acc_s.at[dst_v.at[j]], ssem[b],
                             add=True)

        _gather(0, 0)
        _gather(1, 1)

        @pl.loop(0, KT, step=4)
        def _edges(j0):
            for b in range(4):
                j = j0 + b
                pltpu.make_async_copy(hs_hbm.at[src_v.at[j]], rows_v.at[b],
                                      gsem[b]).wait()
                _scatter(j, b)
                b2 = (b + 2) % 4

                @pl.when(j >= 2)
                def _drain_scatter():
                    pltpu.make_async_copy(rows_v.at[b2],
                                          acc_s.at[dst_v.at[j]],
                                          ssem[b2]).wait()

                @pl.when(j + 2 < KT)
                def _start_next():
                    _gather(j + 2, b2)

        for t in (KT - 2, KT - 1):
            pltpu.make_async_copy(rows_v.at[t % 4], acc_s.at[dst_v.at[0]],
                                  ssem[t % 4]).wait()

    @pl.when(c == 0)
    def _c0():
        _edge_loop(hs0_hbm)

    @pl.when(c == 1)
    def _c1():
        _edge_loop(hs1_hbm)

    plsc.subcore_barrier()

    r0 = s * RPT
    biases = [bias_v[pl.ds(k * 16, 16)] for k in range(F2 // 16)]
    NP = RPT // CH

    def _pull(p):
        return pltpu.async_copy(acc_s.at[pl.ds(r0 + p * CH, CH)],
                                rows_v.at[p % 4], gsem[p % 4])

    pulls = {0: _pull(0)}
    pushes = {}
    for p in range(NP):
        b = p % 4
        pulls[p].wait()
        if p + 1 < NP:
            if p + 1 >= 4:
                pushes[p - 3].wait()
            pulls[p + 1] = _pull(p + 1)

        @pl.loop(0, CH)
        def _rows(r):
            ridx = jnp.zeros((16,), jnp.int32) + (r0 + p * CH + r)
            nrm = plsc.load_gather(norm_v, [ridx])
            for k in range(F2 // 16):
                v = rows_v[b, r, pl.ds(k * 16, 16)]
                rows_v[b, r, pl.ds(k * 16, 16)] = v * nrm + biases[k]

        pushes[p] = pltpu.async_copy(
            rows_v.at[b],
            out_hbm.at[pl.ds(r0 + p * CH, CH), pl.ds(c * F2, F2)], ssem[b])
    for p in range(max(0, NP - 4), NP):
        pushes[p].wait()


def kernel(feat, edge_index, weight, bias):
    src = edge_index[0]
    dst = edge_index[1]
    src2d = src.reshape(NS * KT, CH)
    dst2d = dst.reshape(NS * KT, CH)
    zeros = jnp.zeros((RPT, F2), jnp.float32)

    degp = _deg_kernel_fn()(src).reshape(NW, NRB, 1, RB)
    hs0, hs1, norm3 = _matmul_norm(feat, weight, degp)
    return _scatter_kernel_fn()(hs0, hs1, src2d, dst2d, zeros,
                                norm3.reshape(N), bias)


# speedup vs baseline: 1.3300x; 1.0089x over previous
import functools

import jax
import jax.numpy as jnp
from jax import lax
from jax.experimental import pallas as pl
from jax.experimental.pallas import tpu as pltpu
from jax.experimental.pallas import tpu_sc as plsc

N = 10000
E = 320000
F = 128
F2 = F // 2
NC = 2
NS = 16
NW = NC * NS
EW = E // NW
CH = 125
KT = (E // NS) // CH
RPT = N // NS
RB = 2000
NRB = N // RB


@functools.cache
def _deg_kernel_fn():
    return functools.partial(
        pl.kernel,
        out_type=jax.ShapeDtypeStruct((NW * N,), jnp.float32),
        scratch_types=[
            pltpu.VMEM((EW,), jnp.int32),
            pltpu.VMEM((N,), jnp.float32),
            pltpu.SemaphoreType.DMA,
        ],
        mesh=plsc.VectorSubcoreMesh(core_axis_name="c", subcore_axis_name="s",
                                    num_cores=NC, num_subcores=NS),
        compiler_params=pltpu.CompilerParams(needs_layout_passes=False,
                                             use_tc_tiling_on_sc=False),
    )(_deg_body)


def _deg_body(src_hbm, out_hbm, src_v, hist_v, sem):
    c = lax.axis_index("c")
    s = lax.axis_index("s")
    wid = c * NS + s
    stage = pltpu.async_copy(src_hbm.at[pl.ds(wid * EW, EW)], src_v, sem)

    zeros16 = jnp.zeros((16,), jnp.float32)

    @pl.loop(0, N // 16, unroll=8)
    def _zero(i):
        hist_v[pl.ds(i * 16, 16)] = zeros16

    stage.wait()
    ones16 = jnp.ones((16,), jnp.float32)

    @pl.loop(0, EW // 16, unroll=8)
    def _accum(i):
        idx = src_v[pl.ds(i * 16, 16)]
        plsc.addupdate_scatter(hist_v, [idx], ones16)

    pltpu.sync_copy(hist_v, out_hbm.at[pl.ds(wid * N, N)])


def _mm_body(feat_ref, w_ref, degp_ref, hs0_ref, hs1_ref, norm_ref):
    deg = jnp.sum(degp_ref[:, 0, 0, :], axis=0)
    norm = lax.rsqrt(jnp.maximum(deg, 1.0))
    h = jnp.dot(feat_ref[...], w_ref[...], preferred_element_type=jnp.float32)
    hs = h * norm[:, None]
    hs0_ref[...] = hs[:, :F2]
    hs1_ref[...] = hs[:, F2:]
    norm_ref[...] = norm.reshape(1, 1, RB)


def _matmul_norm(feat, weight, degp):
    return pl.pallas_call(
        _mm_body,
        grid=(NRB,),
        in_specs=[
            pl.BlockSpec((RB, F), lambda i: (i, 0)),
            pl.BlockSpec((F, F), lambda i: (0, 0)),
            pl.BlockSpec((NW, 1, 1, RB), lambda i: (0, i, 0, 0)),
        ],
        out_specs=[
            pl.BlockSpec((RB, F2), lambda i: (i, 0)),
            pl.BlockSpec((RB, F2), lambda i: (i, 0)),
            pl.BlockSpec((1, 1, RB), lambda i: (i, 0, 0)),
        ],
        out_shape=[
            jax.ShapeDtypeStruct((N, F2), jnp.float32),
            jax.ShapeDtypeStruct((N, F2), jnp.float32),
            jax.ShapeDtypeStruct((NRB, 1, RB), jnp.float32),
        ],
    )(feat, weight, degp)


@functools.cache
def _scatter_kernel_fn():
    return functools.partial(
        pl.kernel,
        out_type=jax.ShapeDtypeStruct((N, F), jnp.float32),
        scratch_types=[
            pltpu.VMEM((KT, CH), jnp.int32),
            pltpu.VMEM((KT, CH), jnp.int32),
            pltpu.VMEM((5, CH, F2), jnp.float32),
            pltpu.VMEM((640,), jnp.float32),
            pltpu.VMEM((F2,), jnp.float32),
            pltpu.VMEM_SHARED((N, F2), jnp.float32),
            pltpu.SemaphoreType.DMA,
            pltpu.SemaphoreType.DMA,
            pltpu.SemaphoreType.DMA,
            pltpu.SemaphoreType.DMA,
            pltpu.SemaphoreType.DMA,
            pltpu.SemaphoreType.DMA,
            pltpu.SemaphoreType.DMA,
            pltpu.SemaphoreType.DMA,
            pltpu.SemaphoreType.DMA,
            pltpu.SemaphoreType.DMA,
        ],
        mesh=plsc.VectorSubcoreMesh(core_axis_name="c", subcore_axis_name="s",
                                    num_cores=NC, num_subcores=NS),
        compiler_params=pltpu.CompilerParams(needs_layout_passes=False,
                                             use_tc_tiling_on_sc=False),
    )(_scatter_body)


def _scatter_body(hs0_hbm, hs1_hbm, src_hbm, dst_hbm, zeros_hbm,
                  norm_hbm, bias_hbm, out_hbm,
                  src_v, dst_v, rows_v, norm_v, bias_v, acc_s,
                  gs0, gs1, gs2, gs3, gs4, ss0, ss1, ss2, ss3, ss4):
    c = lax.axis_index("c")
    s = lax.axis_index("s")
    gsem = (gs0, gs1, gs2, gs3, gs4)
    ssem = (ss0, ss1, ss2, ss3, ss4)

    r0 = s * RPT
    nbase = (r0 // 8) * 8
    stages = [
        pltpu.async_copy(zeros_hbm, acc_s.at[pl.ds(s * RPT, RPT)], ssem[0]),
        pltpu.async_copy(src_hbm.at[pl.ds(s * KT, KT)], src_v, gsem[0]),
        pltpu.async_copy(dst_hbm.at[pl.ds(s * KT, KT)], dst_v, gsem[1]),
        pltpu.async_copy(norm_hbm.at[pl.ds(nbase, 640)], norm_v, gsem[2]),
        pltpu.async_copy(bias_hbm.at[pl.ds(c * F2, F2)], bias_v, gsem[3]),
    ]
    for d in stages:
        d.wait()
    plsc.subcore_barrier()

    def _edge_loop(hs_hbm):
        def _gather(j, b):
            pltpu.async_copy(hs_hbm.at[src_v.at[j]], rows_v.at[b], gsem[b])

        def _scatter(j, b):
            pltpu.async_copy(rows_v.at[b], acc_s.at[dst_v.at[j]], ssem[b],
                             add=True)

        _gather(0, 0)
        _gather(1, 1)

        @pl.loop(0, KT, step=5)
        def _edges(j0):
            for b in range(5):
                j = j0 + b
                pltpu.make_async_copy(hs_hbm.at[src_v.at[j]], rows_v.at[b],
                                      gsem[b]).wait()
                _scatter(j, b)
                b2 = (b + 2) % 5

                @pl.when(j >= 3)
                def _drain_scatter():
                    pltpu.make_async_copy(rows_v.at[b2],
                                          acc_s.at[dst_v.at[j]],
                                          ssem[b2]).wait()

                @pl.when(j + 2 < KT)
                def _start_next():
                    _gather(j + 2, b2)

        for t in (KT - 3, KT - 2, KT - 1):
            pltpu.make_async_copy(rows_v.at[t % 5], acc_s.at[dst_v.at[0]],
                                  ssem[t % 5]).wait()

    @pl.when(c == 0)
    def _c0():
        _edge_loop(hs0_hbm)

    @pl.when(c == 1)
    def _c1():
        _edge_loop(hs1_hbm)

    plsc.subcore_barrier()

    noff = r0 - nbase
    biases = [bias_v[pl.ds(k * 16, 16)] for k in range(F2 // 16)]
    NP = RPT // CH

    pulls = [pltpu.async_copy(acc_s.at[pl.ds(r0 + p * CH, CH)],
                              rows_v.at[p], gsem[p]) for p in range(NP)]
    pushes = []
    for p in range(NP):
        pulls[p].wait()

        @pl.loop(0, CH)
        def _rows(r):
            ridx = jnp.zeros((16,), jnp.int32) + (noff + p * CH + r)
            nrm = plsc.load_gather(norm_v, [ridx])
            for k in range(F2 // 16):
                v = rows_v[p, r, pl.ds(k * 16, 16)]
                rows_v[p, r, pl.ds(k * 16, 16)] = v * nrm + biases[k]

        pushes.append(pltpu.async_copy(
            rows_v.at[p],
            out_hbm.at[pl.ds(r0 + p * CH, CH), pl.ds(c * F2, F2)], ssem[p]))
    for d in pushes:
        d.wait()


def kernel(feat, edge_index, weight, bias):
    src = edge_index[0]
    dst = edge_index[1]
    src2d = src.reshape(NS * KT, CH)
    dst2d = dst.reshape(NS * KT, CH)
    zeros = jnp.zeros((RPT, F2), jnp.float32)

    degp = _deg_kernel_fn()(src).reshape(NW, NRB, 1, RB)
    hs0, hs1, norm3 = _matmul_norm(feat, weight, degp)
    norm_pad = jnp.concatenate(
        [norm3.reshape(N), jnp.zeros((640,), jnp.float32)])
    return _scatter_kernel_fn()(hs0, hs1, src2d, dst2d, zeros,
                                norm_pad, bias)


# speedup vs baseline: 1.3502x; 1.0152x over previous
import functools

import jax
import jax.numpy as jnp
from jax import lax
from jax.experimental import pallas as pl
from jax.experimental.pallas import tpu as pltpu
from jax.experimental.pallas import tpu_sc as plsc

N = 10000
E = 320000
F = 128
F2 = F // 2
NC = 2
NS = 16
NW = NC * NS
EW = E // NW
CH = 125
KT = (E // NS) // CH
RPT = N // NS
RB = 2000
NRB = N // RB


@functools.cache
def _deg_kernel_fn():
    return functools.partial(
        pl.kernel,
        out_type=jax.ShapeDtypeStruct((NW * N,), jnp.float32),
        scratch_types=[
            pltpu.VMEM((EW,), jnp.int32),
            pltpu.VMEM((N,), jnp.float32),
            pltpu.SemaphoreType.DMA,
        ],
        mesh=plsc.VectorSubcoreMesh(core_axis_name="c", subcore_axis_name="s",
                                    num_cores=NC, num_subcores=NS),
        compiler_params=pltpu.CompilerParams(needs_layout_passes=False,
                                             use_tc_tiling_on_sc=False),
    )(_deg_body)


def _deg_body(src_hbm, out_hbm, src_v, hist_v, sem):
    c = lax.axis_index("c")
    s = lax.axis_index("s")
    wid = c * NS + s
    stage = pltpu.async_copy(src_hbm.at[pl.ds(wid * EW, EW)], src_v, sem)

    zeros16 = jnp.zeros((16,), jnp.float32)

    @pl.loop(0, N // 16, unroll=8)
    def _zero(i):
        hist_v[pl.ds(i * 16, 16)] = zeros16

    stage.wait()
    ones16 = jnp.ones((16,), jnp.float32)

    @pl.loop(0, EW // 16, unroll=8)
    def _accum(i):
        idx = src_v[pl.ds(i * 16, 16)]
        plsc.addupdate_scatter(hist_v, [idx], ones16)

    pltpu.sync_copy(hist_v, out_hbm.at[pl.ds(wid * N, N)])


def _mm_body(feat_ref, w_ref, degp_ref, hs0_ref, hs1_ref, norm_ref):
    deg = jnp.sum(degp_ref[...], axis=0)
    norm = lax.rsqrt(jnp.maximum(deg, 1.0))
    h = jnp.dot(feat_ref[...], w_ref[...], preferred_element_type=jnp.float32)
    hs = h * norm[:, None]
    hs0_ref[...] = hs[:, :F2]
    hs1_ref[...] = hs[:, F2:]
    norm_ref[...] = norm.reshape(1, N)


def _matmul_norm(feat, weight, degp):
    return pl.pallas_call(
        _mm_body,
        out_shape=[
            jax.ShapeDtypeStruct((N, F2), jnp.float32),
            jax.ShapeDtypeStruct((N, F2), jnp.float32),
            jax.ShapeDtypeStruct((1, N), jnp.float32),
        ],
    )(feat, weight, degp)


@functools.cache
def _scatter_kernel_fn():
    return functools.partial(
        pl.kernel,
        out_type=jax.ShapeDtypeStruct((N, F), jnp.float32),
        scratch_types=[
            pltpu.VMEM((KT, CH), jnp.int32),
            pltpu.VMEM((KT, CH), jnp.int32),
            pltpu.VMEM((5, CH, F2), jnp.float32),
            pltpu.VMEM((640,), jnp.float32),
            pltpu.VMEM((F2,), jnp.float32),
            pltpu.VMEM_SHARED((N, F2), jnp.float32),
            pltpu.SemaphoreType.DMA,
            pltpu.SemaphoreType.DMA,
            pltpu.SemaphoreType.DMA,
            pltpu.SemaphoreType.DMA,
            pltpu.SemaphoreType.DMA,
            pltpu.SemaphoreType.DMA,
            pltpu.SemaphoreType.DMA,
            pltpu.SemaphoreType.DMA,
            pltpu.SemaphoreType.DMA,
            pltpu.SemaphoreType.DMA,
        ],
        mesh=plsc.VectorSubcoreMesh(core_axis_name="c", subcore_axis_name="s",
                                    num_cores=NC, num_subcores=NS),
        compiler_params=pltpu.CompilerParams(needs_layout_passes=False,
                                             use_tc_tiling_on_sc=False),
    )(_scatter_body)


def _scatter_body(hs0_hbm, hs1_hbm, src_hbm, dst_hbm, zeros_hbm,
                  norm_hbm, bias_hbm, out_hbm,
                  src_v, dst_v, rows_v, norm_v, bias_v, acc_s,
                  gs0, gs1, gs2, gs3, gs4, ss0, ss1, ss2, ss3, ss4):
    c = lax.axis_index("c")
    s = lax.axis_index("s")
    gsem = (gs0, gs1, gs2, gs3, gs4)
    ssem = (ss0, ss1, ss2, ss3, ss4)

    r0 = s * RPT
    nbase = (r0 // 8) * 8
    stages = [
        pltpu.async_copy(zeros_hbm, acc_s.at[pl.ds(s * RPT, RPT)], ssem[0]),
        pltpu.async_copy(src_hbm.at[pl.ds(s * KT, KT)], src_v, gsem[0]),
        pltpu.async_copy(dst_hbm.at[pl.ds(s * KT, KT)], dst_v, gsem[1]),
        pltpu.async_copy(norm_hbm.at[pl.ds(nbase, 640)], norm_v, gsem[2]),
        pltpu.async_copy(bias_hbm.at[pl.ds(c * F2, F2)], bias_v, gsem[3]),
    ]
    for d in stages:
        d.wait()
    plsc.subcore_barrier()

    def _edge_loop(hs_hbm):
        def _gather(j, b):
            pltpu.async_copy(hs_hbm.at[src_v.at[j]], rows_v.at[b], gsem[b])

        def _scatter(j, b):
            pltpu.async_copy(rows_v.at[b], acc_s.at[dst_v.at[j]], ssem[b],
                             add=True)

        _gather(0, 0)
        _gather(1, 1)

        @pl.loop(0, KT, step=5)
        def _edges(j0):
            for b in range(5):
                j = j0 + b
                pltpu.make_async_copy(hs_hbm.at[src_v.at[j]], rows_v.at[b],
                                      gsem[b]).wait()
                _scatter(j, b)
                b2 = (b + 2) % 5

                @pl.when(j >= 3)
                def _drain_scatter():
                    pltpu.make_async_copy(rows_v.at[b2],
                                          acc_s.at[dst_v.at[j]],
                                          ssem[b2]).wait()

                @pl.when(j + 2 < KT)
                def _start_next():
                    _gather(j + 2, b2)

        for t in (KT - 3, KT - 2, KT - 1):
            pltpu.make_async_copy(rows_v.at[t % 5], acc_s.at[dst_v.at[0]],
                                  ssem[t % 5]).wait()

    @pl.when(c == 0)
    def _c0():
        _edge_loop(hs0_hbm)

    @pl.when(c == 1)
    def _c1():
        _edge_loop(hs1_hbm)

    plsc.subcore_barrier()

    noff = r0 - nbase
    biases = [bias_v[pl.ds(k * 16, 16)] for k in range(F2 // 16)]
    NP = RPT // CH

    pulls = [pltpu.async_copy(acc_s.at[pl.ds(r0 + p * CH, CH)],
                              rows_v.at[p], gsem[p]) for p in range(NP)]
    pushes = []
    for p in range(NP):
        pulls[p].wait()

        @pl.loop(0, CH)
        def _rows(r):
            ridx = jnp.zeros((16,), jnp.int32) + (noff + p * CH + r)
            nrm = plsc.load_gather(norm_v, [ridx])
            for k in range(F2 // 16):
                v = rows_v[p, r, pl.ds(k * 16, 16)]
                rows_v[p, r, pl.ds(k * 16, 16)] = v * nrm + biases[k]

        pushes.append(pltpu.async_copy(
            rows_v.at[p],
            out_hbm.at[pl.ds(r0 + p * CH, CH), pl.ds(c * F2, F2)], ssem[p]))
    for d in pushes:
        d.wait()


def kernel(feat, edge_index, weight, bias):
    src = edge_index[0]
    dst = edge_index[1]
    src2d = src.reshape(NS * KT, CH)
    dst2d = dst.reshape(NS * KT, CH)
    zeros = jnp.zeros((RPT, F2), jnp.float32)

    degp = _deg_kernel_fn()(src).reshape(NW, N)
    hs0, hs1, norm3 = _matmul_norm(feat, weight, degp)
    norm_pad = jnp.concatenate(
        [norm3.reshape(N), jnp.zeros((640,), jnp.float32)])
    return _scatter_kernel_fn()(hs0, hs1, src2d, dst2d, zeros,
                                norm_pad, bias)


# speedup vs baseline: 1.3828x; 1.0242x over previous
import functools

import jax
import jax.numpy as jnp
from jax import lax
from jax.experimental import pallas as pl
from jax.experimental.pallas import tpu as pltpu
from jax.experimental.pallas import tpu_sc as plsc

N = 10000
E = 320000
F = 128
F2 = F // 2
NC = 2
NS = 16
NW = NC * NS
EW = E // NW
CH = 125
KT = (E // NS) // CH
RPT = N // NS
RB = 2000
NRB = N // RB


@functools.cache
def _deg_kernel_fn():
    return functools.partial(
        pl.kernel,
        out_type=jax.ShapeDtypeStruct((NW * N,), jnp.float32),
        scratch_types=[
            pltpu.VMEM((EW,), jnp.int32),
            pltpu.VMEM((N,), jnp.float32),
            pltpu.SemaphoreType.DMA,
        ],
        mesh=plsc.VectorSubcoreMesh(core_axis_name="c", subcore_axis_name="s",
                                    num_cores=NC, num_subcores=NS),
        compiler_params=pltpu.CompilerParams(needs_layout_passes=False,
                                             use_tc_tiling_on_sc=False),
    )(_deg_body)


def _deg_body(src_hbm, out_hbm, src_v, hist_v, sem):
    c = lax.axis_index("c")
    s = lax.axis_index("s")
    wid = c * NS + s
    stage = pltpu.async_copy(src_hbm.at[pl.ds(wid * EW, EW)], src_v, sem)

    zeros16 = jnp.zeros((16,), jnp.float32)

    @pl.loop(0, N // 16, unroll=8)
    def _zero(i):
        hist_v[pl.ds(i * 16, 16)] = zeros16

    stage.wait()
    ones16 = jnp.ones((16,), jnp.float32)

    @pl.loop(0, EW // 16, unroll=8)
    def _accum(i):
        idx = src_v[pl.ds(i * 16, 16)]
        plsc.addupdate_scatter(hist_v, [idx], ones16)

    pltpu.sync_copy(hist_v, out_hbm.at[pl.ds(wid * N, N)])


def _mm_body(feat_ref, w_ref, degp_ref, src_ref,
             hs_ref, norm_ref, srcx0_ref, srcx1_ref):
    deg = jnp.sum(degp_ref[...], axis=0)
    norm = lax.rsqrt(jnp.maximum(deg, 1.0))
    h = jnp.dot(feat_ref[...], w_ref[...], preferred_element_type=jnp.float32)
    hs_ref[...] = h * norm[:, None]
    norm_ref[...] = norm.reshape(1, N)
    srcx0_ref[...] = src_ref[...] * 2
    srcx1_ref[...] = src_ref[...] * 2 + 1


def _matmul_norm(feat, weight, degp, src2d):
    return pl.pallas_call(
        _mm_body,
        out_shape=[
            jax.ShapeDtypeStruct((N, F), jnp.float32),
            jax.ShapeDtypeStruct((1, N), jnp.float32),
            jax.ShapeDtypeStruct((NS * KT, CH), jnp.int32),
            jax.ShapeDtypeStruct((NS * KT, CH), jnp.int32),
        ],
    )(feat, weight, degp, src2d)


@functools.cache
def _scatter_kernel_fn():
    return functools.partial(
        pl.kernel,
        out_type=jax.ShapeDtypeStruct((N, F), jnp.float32),
        scratch_types=[
            pltpu.VMEM((KT, CH), jnp.int32),
            pltpu.VMEM((KT, CH), jnp.int32),
            pltpu.VMEM((5, CH, F2), jnp.float32),
            pltpu.VMEM((640,), jnp.float32),
            pltpu.VMEM((F2,), jnp.float32),
            pltpu.VMEM_SHARED((N, F2), jnp.float32),
            pltpu.SemaphoreType.DMA,
            pltpu.SemaphoreType.DMA,
            pltpu.SemaphoreType.DMA,
            pltpu.SemaphoreType.DMA,
            pltpu.SemaphoreType.DMA,
            pltpu.SemaphoreType.DMA,
            pltpu.SemaphoreType.DMA,
            pltpu.SemaphoreType.DMA,
            pltpu.SemaphoreType.DMA,
            pltpu.SemaphoreType.DMA,
        ],
        mesh=plsc.VectorSubcoreMesh(core_axis_name="c", subcore_axis_name="s",
                                    num_cores=NC, num_subcores=NS),
        compiler_params=pltpu.CompilerParams(needs_layout_passes=False,
                                             use_tc_tiling_on_sc=False),
    )(_scatter_body)


def _scatter_body(hs2_hbm, srcx0_hbm, srcx1_hbm, dst_hbm, zeros_hbm,
                  norm_hbm, bias_hbm, out_hbm,
                  src_v, dst_v, rows_v, norm_v, bias_v, acc_s,
                  gs0, gs1, gs2, gs3, gs4, ss0, ss1, ss2, ss3, ss4):
    c = lax.axis_index("c")
    s = lax.axis_index("s")
    gsem = (gs0, gs1, gs2, gs3, gs4)
    ssem = (ss0, ss1, ss2, ss3, ss4)

    r0 = s * RPT
    nbase = (r0 // 8) * 8
    stages = [
        pltpu.async_copy(zeros_hbm, acc_s.at[pl.ds(s * RPT, RPT)], ssem[0]),
        pltpu.async_copy(dst_hbm.at[pl.ds(s * KT, KT)], dst_v, gsem[1]),
        pltpu.async_copy(norm_hbm.at[pl.ds(nbase, 640)], norm_v, gsem[2]),
        pltpu.async_copy(bias_hbm.at[pl.ds(c * F2, F2)], bias_v, gsem[3]),
    ]

    @pl.when(c == 0)
    def _stage0():
        pltpu.sync_copy(srcx0_hbm.at[pl.ds(s * KT, KT)], src_v)

    @pl.when(c == 1)
    def _stage1():
        pltpu.sync_copy(srcx1_hbm.at[pl.ds(s * KT, KT)], src_v)

    for d in stages:
        d.wait()
    plsc.subcore_barrier()

    def _edge_loop(hs_hbm):
        def _gather(j, b):
            pltpu.async_copy(hs_hbm.at[src_v.at[j]], rows_v.at[b], gsem[b])

        def _scatter(j, b):
            pltpu.async_copy(rows_v.at[b], acc_s.at[dst_v.at[j]], ssem[b],
                             add=True)

        _gather(0, 0)
        _gather(1, 1)

        @pl.loop(0, KT, step=5)
        def _edges(j0):
            for b in range(5):
                j = j0 + b
                pltpu.make_async_copy(hs_hbm.at[src_v.at[j]], rows_v.at[b],
                                      gsem[b]).wait()
                _scatter(j, b)
                b2 = (b + 2) % 5

                @pl.when(j >= 3)
                def _drain_scatter():
                    pltpu.make_async_copy(rows_v.at[b2],
                                          acc_s.at[dst_v.at[j]],
                                          ssem[b2]).wait()

                @pl.when(j + 2 < KT)
                def _start_next():
                    _gather(j + 2, b2)

        for t in (KT - 3, KT - 2, KT - 1):
            pltpu.make_async_copy(rows_v.at[t % 5], acc_s.at[dst_v.at[0]],
                                  ssem[t % 5]).wait()

    _edge_loop(hs2_hbm)

    plsc.subcore_barrier()

    noff = r0 - nbase
    biases = [bias_v[pl.ds(k * 16, 16)] for k in range(F2 // 16)]
    NP = RPT // CH

    pulls = [pltpu.async_copy(acc_s.at[pl.ds(r0 + p * CH, CH)],
                              rows_v.at[p], gsem[p]) for p in range(NP)]
    pushes = []
    for p in range(NP):
        pulls[p].wait()

        @pl.loop(0, CH)
        def _rows(r):
            ridx = jnp.zeros((16,), jnp.int32) + (noff + p * CH + r)
            nrm = plsc.load_gather(norm_v, [ridx])
            for k in range(F2 // 16):
                v = rows_v[p, r, pl.ds(k * 16, 16)]
                rows_v[p, r, pl.ds(k * 16, 16)] = v * nrm + biases[k]

        pushes.append(pltpu.async_copy(
            rows_v.at[p],
            out_hbm.at[pl.ds(r0 + p * CH, CH), pl.ds(c * F2, F2)], ssem[p]))
    for d in pushes:
        d.wait()


def kernel(feat, edge_index, weight, bias):
    src = edge_index[0]
    dst = edge_index[1]
    src2d = src.reshape(NS * KT, CH)
    dst2d = dst.reshape(NS * KT, CH)
    zeros = jnp.zeros((RPT, F2), jnp.float32)

    degp = _deg_kernel_fn()(src).reshape(NW, N)
    hs, norm3, srcx0, srcx1 = _matmul_norm(feat, weight, degp, src2d)
    hs2 = hs.reshape(2 * N, F2)
    norm_pad = jnp.concatenate(
        [norm3.reshape(N), jnp.zeros((640,), jnp.float32)])
    return _scatter_kernel_fn()(hs2, srcx0, srcx1, dst2d, zeros,
                                norm_pad, bias)
